# baseline (speedup 1.0000x reference)
import jax
from jax.experimental import pallas as pl
from jax.experimental.pallas import tpu as pltpu


_ROWS = 320000
_CHUNKS = [106664, 106664, 106672]
assert sum(_CHUNKS) == _ROWS
_OFFS = [sum(_CHUNKS[:i]) for i in range(len(_CHUNKS))]
_SLOT_ROWS = max(_CHUNKS)


def _copy_kernel(x_hbm, o_hbm, buf, sem):
    for i in range(len(_CHUNKS)):
        sl = pl.ds(_OFFS[i], _CHUNKS[i])
        bsl = pl.ds(0, _CHUNKS[i])
        pltpu.make_async_copy(x_hbm.at[sl], buf.at[bsl], sem).start()
        pltpu.make_async_copy(x_hbm.at[sl], buf.at[bsl], sem).wait()
        pltpu.make_async_copy(buf.at[bsl], o_hbm.at[sl], sem).start()
        pltpu.make_async_copy(buf.at[bsl], o_hbm.at[sl], sem).wait()


def kernel(x_j):
    return pl.pallas_call(
        _copy_kernel,
        out_shape=jax.ShapeDtypeStruct(x_j.shape, x_j.dtype),
        in_specs=[pl.BlockSpec(memory_space=pl.ANY)],
        out_specs=pl.BlockSpec(memory_space=pl.ANY),
        scratch_shapes=[
            pltpu.VMEM((_SLOT_ROWS, 128), jax.numpy.float32),
            pltpu.SemaphoreType.DMA,
        ],
        compiler_params=pltpu.CompilerParams(vmem_limit_bytes=67108864),
    )(x_j)


# speedup vs baseline: 1.0277x; 1.0277x over previous
import jax
from jax.experimental import pallas as pl
from jax.experimental.pallas import tpu as pltpu


_ROWS = 320000
_BLOCK_ROWS = 28000


def _copy_kernel(x_ref, o_ref):
    o_ref[...] = x_ref[...]


def kernel(x_j):
    grid = (pl.cdiv(_ROWS, _BLOCK_ROWS),)
    return pl.pallas_call(
        _copy_kernel,
        out_shape=jax.ShapeDtypeStruct(x_j.shape, x_j.dtype),
        grid=grid,
        in_specs=[pl.BlockSpec((_BLOCK_ROWS, 128), lambda i: (i, 0))],
        out_specs=pl.BlockSpec((_BLOCK_ROWS, 128), lambda i: (i, 0)),
        compiler_params=pltpu.CompilerParams(vmem_limit_bytes=67108864),
    )(x_j)
